# baseline (speedup 1.0000x reference)
import functools

import jax
import jax.numpy as jnp
from jax import lax
from jax.experimental import pallas as pl
from jax.experimental.pallas import tpu as pltpu
from jax.experimental.pallas import tpu_sc as plsc

_VOCAB = 1000000
_DIM = 64
_OUT_DIM = 5
_SEQ = 200
_BATCH = 4096

_NC = 2
_NS = 16
_NW = _NC * _NS
_BPW = _BATCH // _NW
_LANES = 16
_PW = 8
_VBLK = 8192
_DEPTH = 8


def _project_body(embt_ref, w_ref, p_ref):
    w = w_ref[...] * (1.0 / _SEQ)
    mm = lax.dot_general(embt_ref[...], w, (((0,), (1,)), ((), ())),
                         preferred_element_type=jnp.float32)
    p_ref[:, 0:_PW] = mm


def _project(embt, w8):
    grid = (_VOCAB + _VBLK - 1) // _VBLK
    return pl.pallas_call(
        _project_body,
        grid=(grid,),
        in_specs=[
            pl.BlockSpec((_DIM, _VBLK), lambda i: (0, i)),
            pl.BlockSpec((_PW, _DIM), lambda i: (0, 0)),
        ],
        out_specs=pl.BlockSpec((_VBLK, 128), lambda i: (i, 0)),
        out_shape=jax.ShapeDtypeStruct((_VOCAB, 128), jnp.float32),
    )(embt, w8)


_NCH = 8


def _sc_pool_body(text_hbm, p_hbm, out_hbm, idx_v, idx8_v, pools, sems):
    wid = lax.axis_index("s") * _NC + lax.axis_index("c")
    base = wid * _BPW

    pltpu.sync_copy(text_hbm.at[:, pl.ds(base, _BPW)], idx_v)

    three = jnp.full((_LANES,), 3, jnp.int32)

    def shift_body(s, carry):
        for c in range(_BPW // _LANES):
            sl = pl.ds(c * _LANES, _LANES)
            idx8_v[s, sl] = lax.shift_left(idx_v[s, sl], three)
        return carry

    lax.fori_loop(0, _SEQ, shift_body, 0)

    def fire(s, k, add=True):
        pltpu.async_copy(p_hbm.at[idx8_v.at[s]], pools[k], sems[k], add=add)

    def drain(k):
        pltpu.make_async_copy(p_hbm.at[idx8_v.at[0]], pools[k], sems[k]).wait()

    for k in range(_NCH):
        fire(k, k, add=False)
    for k in range(_NCH):
        drain(k)
        fire(k + _NCH, k)

    def body(r, carry):
        s0 = (r + 2) * _NCH
        for k in range(_NCH):
            drain(k)

            @pl.when(s0 + k < _SEQ)
            def _():
                fire(s0 + k, k)

        return carry

    rounds = _SEQ // _NCH - 1
    lax.fori_loop(0, rounds, body, 0)

    for k in range(_NCH):
        pltpu.sync_copy(pools[k], out_hbm.at[k, pl.ds(base, _BPW), :])


@functools.partial(
    pl.kernel,
    out_type=jax.ShapeDtypeStruct((_NCH, _BATCH, 2 * _PW), jnp.float32),
    mesh=plsc.VectorSubcoreMesh(core_axis_name="c", subcore_axis_name="s"),
    compiler_params=pltpu.CompilerParams(use_tc_tiling_on_sc=False),
    scratch_types=[
        pltpu.VMEM((_SEQ, _BPW), jnp.int32),
        pltpu.VMEM((_SEQ, _BPW), jnp.int32),
        [pltpu.VMEM((_BPW, 2 * _PW), jnp.float32) for _ in range(_NCH)],
        [pltpu.SemaphoreType.DMA for _ in range(_NCH)],
    ],
)
def _sc_pool(text_hbm, p_hbm, out_hbm, idx_v, idx8_v, pools, sems):
    _sc_pool_body(text_hbm, p_hbm, out_hbm, idx_v, idx8_v, pools, sems)


def _finish_body(p_ref, b_ref, o_ref):
    pooled = jnp.sum(p_ref[...], axis=0)
    o_ref[...] = pooled[:, :_OUT_DIM] + b_ref[...]


def kernel(text, emb, W, b):
    text = text.astype(jnp.int32)
    w8 = jnp.zeros((_PW, _DIM), jnp.float32).at[:_OUT_DIM].set(W)
    proj = _project(emb.T, w8)
    sums = _sc_pool(text, proj.reshape(8 * _VOCAB, 2 * _PW))
    out = pl.pallas_call(
        _finish_body,
        out_shape=jax.ShapeDtypeStruct((_BATCH, _OUT_DIM), jnp.float32),
    )(sums, b.reshape(1, _OUT_DIM))
    return out


# speedup vs baseline: 1.0962x; 1.0962x over previous
import functools

import jax
import jax.numpy as jnp
from jax import lax
from jax.experimental import pallas as pl
from jax.experimental.pallas import tpu as pltpu
from jax.experimental.pallas import tpu_sc as plsc

_VOCAB = 1000000
_DIM = 64
_OUT_DIM = 5
_SEQ = 200
_BATCH = 4096

_NC = 2
_NS = 16
_NW = _NC * _NS
_BPW = _BATCH // _NW
_LANES = 16
_PW = 8
_VBLK = 16384
_GRID = (_VOCAB + _VBLK - 1) // _VBLK
_DEPTH = 8


def _project_body(embt_ref, w_ref, p_ref):
    w = w_ref[...] * (1.0 / _SEQ)
    mm = lax.dot_general(embt_ref[...], w, (((0,), (1,)), ((), ())),
                         preferred_element_type=jnp.float32)
    p_ref[:, 0:_PW] = mm


def _project(embt, w8):
    return pl.pallas_call(
        _project_body,
        grid=(_GRID,),
        in_specs=[
            pl.BlockSpec((_DIM, _VBLK), lambda i: (0, i)),
            pl.BlockSpec((_PW, _DIM), lambda i: (0, 0)),
        ],
        out_specs=pl.BlockSpec((_VBLK, 128), lambda i: (i, 0)),
        out_shape=jax.ShapeDtypeStruct((_VOCAB, 128), jnp.float32),
    )(embt, w8)


def _sc_pool_body(text_hbm, p_hbm, out_hbm, idx_v, idx8_v, pool_v, sem):
    wid = lax.axis_index("s") * _NC + lax.axis_index("c")
    base = wid * _BPW

    pltpu.sync_copy(text_hbm.at[:, pl.ds(base, _BPW)], idx_v)

    three = jnp.full((_LANES,), 3, jnp.int32)

    def shift_body(s, carry):
        for c in range(_BPW // _LANES):
            sl = pl.ds(c * _LANES, _LANES)
            idx8_v[s, sl] = lax.shift_left(idx_v[s, sl], three)
        return carry

    lax.fori_loop(0, _SEQ, shift_body, 0)

    def fire(s, add=True):
        pltpu.async_copy(p_hbm.at[idx8_v.at[s]], pool_v, sem, add=add)

    def drain():
        pltpu.make_async_copy(p_hbm.at[idx8_v.at[0]], pool_v, sem).wait()

    fire(0, add=False)
    drain()
    for s in range(1, _DEPTH + 1):
        fire(s)

    def body(p, carry):
        drain()

        @pl.when(p + _DEPTH + 1 < _SEQ)
        def _():
            fire(p + _DEPTH + 1)

        return carry

    lax.fori_loop(0, _SEQ - 1, body, 0)

    pltpu.sync_copy(pool_v, out_hbm.at[pl.ds(base, _BPW), :])


@functools.partial(
    pl.kernel,
    out_type=jax.ShapeDtypeStruct((_BATCH, 2 * _PW), jnp.float32),
    mesh=plsc.VectorSubcoreMesh(core_axis_name="c", subcore_axis_name="s"),
    compiler_params=pltpu.CompilerParams(use_tc_tiling_on_sc=False),
    scratch_types=[
        pltpu.VMEM((_SEQ, _BPW), jnp.int32),
        pltpu.VMEM((_SEQ, _BPW), jnp.int32),
        pltpu.VMEM((_BPW, 2 * _PW), jnp.float32),
        pltpu.SemaphoreType.DMA,
    ],
)
def _sc_pool(text_hbm, p_hbm, out_hbm, idx_v, idx8_v, pool_v, sem):
    _sc_pool_body(text_hbm, p_hbm, out_hbm, idx_v, idx8_v, pool_v, sem)


def _finish_body(p_ref, b_ref, o_ref):
    o_ref[...] = p_ref[:, :_OUT_DIM] + b_ref[...]


def kernel(text, emb, W, b):
    text = text.astype(jnp.int32)
    w8 = jnp.zeros((_PW, _DIM), jnp.float32).at[:_OUT_DIM].set(W)
    proj = _project(emb.T, w8)
    sums = _sc_pool(text, proj.reshape(8 * _VOCAB, 2 * _PW))
    out = pl.pallas_call(
        _finish_body,
        out_shape=jax.ShapeDtypeStruct((_BATCH, _OUT_DIM), jnp.float32),
    )(sums, b.reshape(1, _OUT_DIM))
    return out


# speedup vs baseline: 1.1195x; 1.0213x over previous
import functools

import jax
import jax.numpy as jnp
from jax import lax
from jax.experimental import pallas as pl
from jax.experimental.pallas import tpu as pltpu
from jax.experimental.pallas import tpu_sc as plsc

_VOCAB = 1000000
_DIM = 64
_OUT_DIM = 5
_SEQ = 200
_BATCH = 4096

_NC = 2
_NS = 16
_NW = _NC * _NS
_BPW = _BATCH // _NW
_LANES = 16
_PW = 8
_VBLK = 32768
_GRID = (_VOCAB + _VBLK - 1) // _VBLK
_DEPTH = 8


def _project_body(embt_ref, w_ref, p_ref):
    w = w_ref[...] * (1.0 / _SEQ)
    mm = lax.dot_general(embt_ref[...], w, (((0,), (1,)), ((), ())),
                         preferred_element_type=jnp.float32)
    p_ref[:, 0:_PW] = mm


def _project(embt, w8):
    return pl.pallas_call(
        _project_body,
        grid=(_GRID,),
        in_specs=[
            pl.BlockSpec((_DIM, _VBLK), lambda i: (0, i)),
            pl.BlockSpec((_PW, _DIM), lambda i: (0, 0)),
        ],
        out_specs=pl.BlockSpec((_VBLK, 128), lambda i: (i, 0)),
        out_shape=jax.ShapeDtypeStruct((_VOCAB, 128), jnp.float32),
    )(embt, w8)


def _sc_pool_body(text_hbm, p_hbm, out_hbm, idx_v, idx8_v, pool_v, sem):
    wid = lax.axis_index("s") * _NC + lax.axis_index("c")
    base = wid * _BPW

    pltpu.sync_copy(text_hbm.at[:, pl.ds(base, _BPW)], idx_v)

    three = jnp.full((_LANES,), 3, jnp.int32)

    def shift_body(s, carry):
        for c in range(_BPW // _LANES):
            sl = pl.ds(c * _LANES, _LANES)
            idx8_v[s, sl] = lax.shift_left(idx_v[s, sl], three)
        return carry

    lax.fori_loop(0, _SEQ, shift_body, 0)

    def fire(s, add=True):
        pltpu.async_copy(p_hbm.at[idx8_v.at[s]], pool_v, sem, add=add)

    def drain():
        pltpu.make_async_copy(p_hbm.at[idx8_v.at[0]], pool_v, sem).wait()

    fire(0, add=False)
    drain()
    for s in range(1, _DEPTH + 1):
        fire(s)

    def body(p, carry):
        drain()

        @pl.when(p + _DEPTH + 1 < _SEQ)
        def _():
            fire(p + _DEPTH + 1)

        return carry

    lax.fori_loop(0, _SEQ - 1, body, 0)

    pltpu.sync_copy(pool_v, out_hbm.at[pl.ds(base, _BPW), :])


@functools.partial(
    pl.kernel,
    out_type=jax.ShapeDtypeStruct((_BATCH, 2 * _PW), jnp.float32),
    mesh=plsc.VectorSubcoreMesh(core_axis_name="c", subcore_axis_name="s"),
    compiler_params=pltpu.CompilerParams(use_tc_tiling_on_sc=False),
    scratch_types=[
        pltpu.VMEM((_SEQ, _BPW), jnp.int32),
        pltpu.VMEM((_SEQ, _BPW), jnp.int32),
        pltpu.VMEM((_BPW, 2 * _PW), jnp.float32),
        pltpu.SemaphoreType.DMA,
    ],
)
def _sc_pool(text_hbm, p_hbm, out_hbm, idx_v, idx8_v, pool_v, sem):
    _sc_pool_body(text_hbm, p_hbm, out_hbm, idx_v, idx8_v, pool_v, sem)


def _finish_body(p_ref, b_ref, o_ref):
    o_ref[...] = p_ref[:, :_OUT_DIM] + b_ref[...]


def kernel(text, emb, W, b):
    text = text.astype(jnp.int32)
    w8 = jnp.zeros((_PW, _DIM), jnp.float32).at[:_OUT_DIM].set(W)
    proj = _project(emb.T, w8)
    sums = _sc_pool(text, proj.reshape(8 * _VOCAB, 2 * _PW))
    out = pl.pallas_call(
        _finish_body,
        out_shape=jax.ShapeDtypeStruct((_BATCH, _OUT_DIM), jnp.float32),
    )(sums, b.reshape(1, _OUT_DIM))
    return out


# speedup vs baseline: 1.1467x; 1.0243x over previous
import functools

import jax
import jax.numpy as jnp
from jax import lax
from jax.experimental import pallas as pl
from jax.experimental.pallas import tpu as pltpu
from jax.experimental.pallas import tpu_sc as plsc

_VOCAB = 1000000
_DIM = 64
_OUT_DIM = 5
_SEQ = 200
_BATCH = 4096

_NC = 2
_NS = 16
_NW = _NC * _NS
_BPW = _BATCH // _NW
_LANES = 16
_PW = 8
_VBLK = 32768
_GRID = (_VOCAB + _VBLK - 1) // _VBLK
_DEPTH = 16


def _project_body(embt_ref, w_ref, p_ref):
    w = w_ref[...] * (1.0 / _SEQ)
    mm = lax.dot_general(embt_ref[...], w, (((0,), (1,)), ((), ())),
                         preferred_element_type=jnp.float32)
    p_ref[:, 0:_PW] = mm


def _project(embt, w8):
    return pl.pallas_call(
        _project_body,
        grid=(_GRID,),
        in_specs=[
            pl.BlockSpec((_DIM, _VBLK), lambda i: (0, i)),
            pl.BlockSpec((_PW, _DIM), lambda i: (0, 0)),
        ],
        out_specs=pl.BlockSpec((_VBLK, 128), lambda i: (i, 0)),
        out_shape=jax.ShapeDtypeStruct((_VOCAB, 128), jnp.float32),
    )(embt, w8)


def _sc_pool_body(text_hbm, p_hbm, out_hbm, idx_v, idx8_v, pool_v, sem):
    wid = lax.axis_index("s") * _NC + lax.axis_index("c")
    base = wid * _BPW

    pltpu.sync_copy(text_hbm.at[:, pl.ds(base, _BPW)], idx_v)

    three = jnp.full((_LANES,), 3, jnp.int32)

    def shift_body(s, carry):
        for c in range(_BPW // _LANES):
            sl = pl.ds(c * _LANES, _LANES)
            idx8_v[s, sl] = lax.shift_left(idx_v[s, sl], three)
        return carry

    lax.fori_loop(0, _SEQ, shift_body, 0)

    def fire(s, add=True):
        pltpu.async_copy(p_hbm.at[idx8_v.at[s]], pool_v, sem, add=add)

    def drain():
        pltpu.make_async_copy(p_hbm.at[idx8_v.at[0]], pool_v, sem).wait()

    fire(0, add=False)
    drain()
    for s in range(1, _DEPTH + 1):
        fire(s)

    def body(p, carry):
        drain()

        @pl.when(p + _DEPTH + 1 < _SEQ)
        def _():
            fire(p + _DEPTH + 1)

        return carry

    lax.fori_loop(0, _SEQ - 1, body, 0)

    pltpu.sync_copy(pool_v, out_hbm.at[pl.ds(base, _BPW), :])


@functools.partial(
    pl.kernel,
    out_type=jax.ShapeDtypeStruct((_BATCH, 2 * _PW), jnp.float32),
    mesh=plsc.VectorSubcoreMesh(core_axis_name="c", subcore_axis_name="s"),
    compiler_params=pltpu.CompilerParams(use_tc_tiling_on_sc=False),
    scratch_types=[
        pltpu.VMEM((_SEQ, _BPW), jnp.int32),
        pltpu.VMEM((_SEQ, _BPW), jnp.int32),
        pltpu.VMEM((_BPW, 2 * _PW), jnp.float32),
        pltpu.SemaphoreType.DMA,
    ],
)
def _sc_pool(text_hbm, p_hbm, out_hbm, idx_v, idx8_v, pool_v, sem):
    _sc_pool_body(text_hbm, p_hbm, out_hbm, idx_v, idx8_v, pool_v, sem)


def _finish_body(p_ref, b_ref, o_ref):
    o_ref[...] = p_ref[:, :_OUT_DIM] + b_ref[...]


def kernel(text, emb, W, b):
    text = text.astype(jnp.int32)
    w8 = jnp.zeros((_PW, _DIM), jnp.float32).at[:_OUT_DIM].set(W)
    proj = _project(emb.T, w8)
    sums = _sc_pool(text, proj.reshape(8 * _VOCAB, 2 * _PW))
    out = pl.pallas_call(
        _finish_body,
        out_shape=jax.ShapeDtypeStruct((_BATCH, _OUT_DIM), jnp.float32),
    )(sums, b.reshape(1, _OUT_DIM))
    return out


# speedup vs baseline: 1.1473x; 1.0005x over previous
import functools

import jax
import jax.numpy as jnp
from jax import lax
from jax.experimental import pallas as pl
from jax.experimental.pallas import tpu as pltpu
from jax.experimental.pallas import tpu_sc as plsc

_VOCAB = 1000000
_DIM = 64
_OUT_DIM = 5
_SEQ = 200
_BATCH = 4096

_NC = 2
_NS = 16
_NW = _NC * _NS
_BPW = _BATCH // _NW
_LANES = 16
_PW = 8
_VBLK = 32768
_GRID = (_VOCAB + _VBLK - 1) // _VBLK
_DEPTH = 32


def _project_body(embt_ref, w_ref, p_ref):
    w = w_ref[...] * (1.0 / _SEQ)
    mm = lax.dot_general(embt_ref[...], w, (((0,), (1,)), ((), ())),
                         preferred_element_type=jnp.float32)
    p_ref[:, 0:_PW] = mm


def _project(embt, w8):
    return pl.pallas_call(
        _project_body,
        grid=(_GRID,),
        in_specs=[
            pl.BlockSpec((_DIM, _VBLK), lambda i: (0, i)),
            pl.BlockSpec((_PW, _DIM), lambda i: (0, 0)),
        ],
        out_specs=pl.BlockSpec((_VBLK, 128), lambda i: (i, 0)),
        out_shape=jax.ShapeDtypeStruct((_VOCAB, 128), jnp.float32),
    )(embt, w8)


def _sc_pool_body(text_hbm, p_hbm, out_hbm, idx_v, idx8_v, pool_v, sem):
    wid = lax.axis_index("s") * _NC + lax.axis_index("c")
    base = wid * _BPW

    pltpu.sync_copy(text_hbm.at[:, pl.ds(base, _BPW)], idx_v)

    three = jnp.full((_LANES,), 3, jnp.int32)

    def shift_body(s, carry):
        for c in range(_BPW // _LANES):
            sl = pl.ds(c * _LANES, _LANES)
            idx8_v[s, sl] = lax.shift_left(idx_v[s, sl], three)
        return carry

    lax.fori_loop(0, _SEQ, shift_body, 0)

    def fire(s, add=True):
        pltpu.async_copy(p_hbm.at[idx8_v.at[s]], pool_v, sem, add=add)

    def drain():
        pltpu.make_async_copy(p_hbm.at[idx8_v.at[0]], pool_v, sem).wait()

    fire(0, add=False)
    drain()
    for s in range(1, _DEPTH + 1):
        fire(s)

    def body(p, carry):
        drain()

        @pl.when(p + _DEPTH + 1 < _SEQ)
        def _():
            fire(p + _DEPTH + 1)

        return carry

    lax.fori_loop(0, _SEQ - 1, body, 0)

    pltpu.sync_copy(pool_v, out_hbm.at[pl.ds(base, _BPW), :])


@functools.partial(
    pl.kernel,
    out_type=jax.ShapeDtypeStruct((_BATCH, 2 * _PW), jnp.float32),
    mesh=plsc.VectorSubcoreMesh(core_axis_name="c", subcore_axis_name="s"),
    compiler_params=pltpu.CompilerParams(use_tc_tiling_on_sc=False),
    scratch_types=[
        pltpu.VMEM((_SEQ, _BPW), jnp.int32),
        pltpu.VMEM((_SEQ, _BPW), jnp.int32),
        pltpu.VMEM((_BPW, 2 * _PW), jnp.float32),
        pltpu.SemaphoreType.DMA,
    ],
)
def _sc_pool(text_hbm, p_hbm, out_hbm, idx_v, idx8_v, pool_v, sem):
    _sc_pool_body(text_hbm, p_hbm, out_hbm, idx_v, idx8_v, pool_v, sem)


def _finish_body(p_ref, b_ref, o_ref):
    o_ref[...] = p_ref[:, :_OUT_DIM] + b_ref[...]


def kernel(text, emb, W, b):
    text = text.astype(jnp.int32)
    w8 = jnp.zeros((_PW, _DIM), jnp.float32).at[:_OUT_DIM].set(W)
    proj = _project(emb.T, w8)
    sums = _sc_pool(text, proj.reshape(8 * _VOCAB, 2 * _PW))
    out = pl.pallas_call(
        _finish_body,
        out_shape=jax.ShapeDtypeStruct((_BATCH, _OUT_DIM), jnp.float32),
    )(sums, b.reshape(1, _OUT_DIM))
    return out
